# Initial kernel scaffold; baseline (speedup 1.0000x reference)
import jax
import jax.numpy as jnp
from jax.experimental import pallas as pl


def kernel(h, edge_index, W, b):
    raise NotImplementedError("write your pallas kernel here")


# speedup vs baseline: 31.3026x; 31.3026x over previous
import functools

import jax
import jax.numpy as jnp
from jax import lax
from jax.experimental import pallas as pl
from jax.experimental.pallas import tpu as pltpu
from jax.experimental.pallas import tpu_sc as plsc

N_NODES = 10000
N_EDGES = 320000
D_FEAT = 128
L = 16
NC, NS = 2, 16
NW = NC * NS
E_PER_W = N_EDGES // NW


def _tc_node_scores(w_ref, h_ref, bvec_ref, o_ref):
    o_ref[...] = (
        lax.dot_general(
            w_ref[...], h_ref[...],
            dimension_numbers=(((1,), (1,)), ((), ())),
            preferred_element_type=jnp.float32,
        )
        + bvec_ref[...]
    )


_sc_mesh = plsc.VectorSubcoreMesh(core_axis_name="c", subcore_axis_name="s")


@functools.partial(
    pl.kernel,
    mesh=_sc_mesh,
    out_type=jax.ShapeDtypeStruct((N_EDGES,), jnp.float32),
    compiler_params=pltpu.CompilerParams(needs_layout_passes=False),
    scratch_types=[
        pltpu.VMEM((N_NODES,), jnp.float32),
        pltpu.VMEM((N_NODES,), jnp.float32),
        pltpu.VMEM((E_PER_W,), jnp.int32),
        pltpu.VMEM((E_PER_W,), jnp.int32),
        pltpu.VMEM((E_PER_W,), jnp.float32),
    ],
)
def _sc_edge_gather(s_hbm, src_hbm, dst_hbm, out_hbm,
                    s1_v, s2_v, src_v, dst_v, out_v):
    wid = lax.axis_index("s") * NC + lax.axis_index("c")
    base = wid * E_PER_W
    pltpu.sync_copy(s_hbm.at[0], s1_v)
    pltpu.sync_copy(s_hbm.at[1], s2_v)
    pltpu.sync_copy(src_hbm.at[pl.ds(base, E_PER_W)], src_v)
    pltpu.sync_copy(dst_hbm.at[pl.ds(base, E_PER_W)], dst_v)

    def body(i, carry):
        off = i * L
        si = src_v[pl.ds(off, L)]
        di = dst_v[pl.ds(off, L)]
        v1 = plsc.load_gather(s1_v, [si])
        v2 = plsc.load_gather(s2_v, [di])
        out_v[pl.ds(off, L)] = v1 + v2
        return carry

    lax.fori_loop(0, E_PER_W // L, body, 0)
    pltpu.sync_copy(out_v, out_hbm.at[pl.ds(base, E_PER_W)])


def kernel(h, edge_index, W, b):
    wmat = W.reshape(2, D_FEAT)
    bvec = jnp.concatenate([jnp.zeros_like(b), b]).reshape(2, 1)
    s = pl.pallas_call(
        _tc_node_scores,
        out_shape=jax.ShapeDtypeStruct((2, N_NODES), jnp.float32),
    )(wmat, h, bvec)
    src = edge_index[0]
    dst = edge_index[1]
    scores = _sc_edge_gather(s, src, dst)
    return scores.reshape(N_EDGES, 1)


# speedup vs baseline: 34.3790x; 1.0983x over previous
import functools

import jax
import jax.numpy as jnp
from jax import lax
from jax.experimental import pallas as pl
from jax.experimental.pallas import tpu as pltpu
from jax.experimental.pallas import tpu_sc as plsc

N_NODES = 10000
N_EDGES = 320000
D_FEAT = 128
L = 16
NC, NS = 2, 16
NW = NC * NS
E_PER_W = N_EDGES // NW


def _tc_node_scores(w_ref, h_ref, bvec_ref, o_ref):
    o_ref[...] = (
        lax.dot_general(
            w_ref[...], h_ref[...],
            dimension_numbers=(((1,), (1,)), ((), ())),
            preferred_element_type=jnp.float32,
        )
        + bvec_ref[...]
    )


_sc_mesh = plsc.VectorSubcoreMesh(core_axis_name="c", subcore_axis_name="s")


@functools.partial(
    pl.kernel,
    mesh=_sc_mesh,
    out_type=jax.ShapeDtypeStruct((N_EDGES,), jnp.float32),
    compiler_params=pltpu.CompilerParams(needs_layout_passes=False),
    scratch_types=[
        pltpu.VMEM((N_NODES,), jnp.float32),
        pltpu.VMEM((N_NODES,), jnp.float32),
        pltpu.VMEM((E_PER_W,), jnp.int32),
        pltpu.VMEM((E_PER_W,), jnp.int32),
        pltpu.VMEM((E_PER_W,), jnp.float32),
        pltpu.SemaphoreType.DMA,
    ],
)
def _sc_edge_gather(s_hbm, src_hbm, dst_hbm, out_hbm,
                    s1_v, s2_v, src_v, dst_v, out_v, sem):
    wid = lax.axis_index("s") * NC + lax.axis_index("c")
    base = wid * E_PER_W
    c1 = pltpu.async_copy(s_hbm.at[0], s1_v, sem)
    c2 = pltpu.async_copy(s_hbm.at[1], s2_v, sem)
    c3 = pltpu.async_copy(src_hbm.at[pl.ds(base, E_PER_W)], src_v, sem)
    c4 = pltpu.async_copy(dst_hbm.at[pl.ds(base, E_PER_W)], dst_v, sem)
    c1.wait()
    c2.wait()
    c3.wait()
    c4.wait()

    @plsc.parallel_loop(0, E_PER_W, L, unroll=8)
    def _body(off):
        si = src_v[pl.ds(off, L)]
        di = dst_v[pl.ds(off, L)]
        v1 = plsc.load_gather(s1_v, [si])
        v2 = plsc.load_gather(s2_v, [di])
        out_v[pl.ds(off, L)] = v1 + v2

    pltpu.sync_copy(out_v, out_hbm.at[pl.ds(base, E_PER_W)])


def kernel(h, edge_index, W, b):
    wmat = W.reshape(2, D_FEAT)
    bvec = jnp.concatenate([jnp.zeros_like(b), b]).reshape(2, 1)
    s = pl.pallas_call(
        _tc_node_scores,
        out_shape=jax.ShapeDtypeStruct((2, N_NODES), jnp.float32),
    )(wmat, h, bvec)
    scores = _sc_edge_gather(s, edge_index[0], edge_index[1])
    return scores.reshape(N_EDGES, 1)


# speedup vs baseline: 48.2902x; 1.4046x over previous
import functools

import jax
import jax.numpy as jnp
from jax import lax
from jax.experimental import pallas as pl
from jax.experimental.pallas import tpu as pltpu
from jax.experimental.pallas import tpu_sc as plsc

N_NODES = 10000
N_EDGES = 320000
D_FEAT = 128
L = 16
NC, NS = 2, 16
NW = NC * NS

TILE = 128
N_TILES = N_EDGES // TILE
T_PER_W = N_TILES // NW
CHUNK = T_PER_W * TILE
REM_BASE = NW * CHUNK
N_REM = N_TILES - NW * T_PER_W


def _tc_node_scores(w_ref, h_ref, bvec_ref, o_ref):
    o_ref[...] = (
        lax.dot_general(
            w_ref[...], h_ref[...],
            dimension_numbers=(((1,), (1,)), ((), ())),
            preferred_element_type=jnp.float32,
        )
        + bvec_ref[...]
    )


_sc_mesh = plsc.VectorSubcoreMesh(core_axis_name="c", subcore_axis_name="s")


@functools.partial(
    pl.kernel,
    mesh=_sc_mesh,
    out_type=jax.ShapeDtypeStruct((N_EDGES,), jnp.float32),
    compiler_params=pltpu.CompilerParams(needs_layout_passes=False),
    scratch_types=[
        pltpu.VMEM((N_NODES,), jnp.float32),
        pltpu.VMEM((N_NODES,), jnp.float32),
        pltpu.VMEM((2, CHUNK), jnp.int32),
        pltpu.VMEM((2, TILE), jnp.int32),
        pltpu.VMEM((CHUNK,), jnp.float32),
        pltpu.VMEM((TILE,), jnp.float32),
        pltpu.SemaphoreType.DMA,
    ],
)
def _sc_edge_gather(s_hbm, edge_hbm, out_hbm,
                    s1_v, s2_v, ei_v, ei2_v, out_v, out2_v, sem):
    wid = lax.axis_index("s") * NC + lax.axis_index("c")
    base = wid * CHUNK
    c1 = pltpu.async_copy(s_hbm.at[0], s1_v, sem)
    c2 = pltpu.async_copy(s_hbm.at[1], s2_v, sem)
    c3 = pltpu.async_copy(edge_hbm.at[:, pl.ds(base, CHUNK)], ei_v, sem)
    c1.wait()
    c2.wait()
    c3.wait()

    @plsc.parallel_loop(0, CHUNK, L, unroll=8)
    def _body(off):
        si = ei_v[0, pl.ds(off, L)]
        di = ei_v[1, pl.ds(off, L)]
        v1 = plsc.load_gather(s1_v, [si])
        v2 = plsc.load_gather(s2_v, [di])
        out_v[pl.ds(off, L)] = v1 + v2

    pltpu.sync_copy(out_v, out_hbm.at[pl.ds(base, CHUNK)])

    @pl.when(wid < N_REM)
    def _rem():
        rbase = REM_BASE + wid * TILE
        pltpu.sync_copy(edge_hbm.at[:, pl.ds(rbase, TILE)], ei2_v)

        @plsc.parallel_loop(0, TILE, L, unroll=8)
        def _body2(off):
            si = ei2_v[0, pl.ds(off, L)]
            di = ei2_v[1, pl.ds(off, L)]
            v1 = plsc.load_gather(s1_v, [si])
            v2 = plsc.load_gather(s2_v, [di])
            out2_v[pl.ds(off, L)] = v1 + v2

        pltpu.sync_copy(out2_v, out_hbm.at[pl.ds(rbase, TILE)])


def kernel(h, edge_index, W, b):
    wmat = W.reshape(2, D_FEAT)
    bvec = jnp.concatenate([jnp.zeros_like(b), b]).reshape(2, 1)
    s = pl.pallas_call(
        _tc_node_scores,
        out_shape=jax.ShapeDtypeStruct((2, N_NODES), jnp.float32),
    )(wmat, h, bvec)
    scores = _sc_edge_gather(s, edge_index)
    return scores.reshape(N_EDGES, 1)


# speedup vs baseline: 49.2984x; 1.0209x over previous
import functools

import jax
import jax.numpy as jnp
from jax import lax
from jax.experimental import pallas as pl
from jax.experimental.pallas import tpu as pltpu
from jax.experimental.pallas import tpu_sc as plsc

N_NODES = 10000
N_EDGES = 320000
D_FEAT = 128
L = 16
NC, NS = 2, 16
NW = NC * NS

TILE = 128
N_TILES = N_EDGES // TILE
T_PER_W = N_TILES // NW
CHUNK = T_PER_W * TILE
REM_BASE = NW * CHUNK
N_REM = N_TILES - NW * T_PER_W


def _tc_node_scores(b_ref, w_ref, h_ref, o_ref):
    rows = lax.broadcasted_iota(jnp.int32, (2, N_NODES), 0)
    bias = jnp.where(rows == 1, b_ref[0, 0], 0.0)
    o_ref[...] = (
        lax.dot_general(
            w_ref[...], h_ref[...],
            dimension_numbers=(((1,), (1,)), ((), ())),
            preferred_element_type=jnp.float32,
        )
        + bias
    )


_sc_mesh = plsc.VectorSubcoreMesh(core_axis_name="c", subcore_axis_name="s")


@functools.partial(
    pl.kernel,
    mesh=_sc_mesh,
    out_type=jax.ShapeDtypeStruct((N_EDGES,), jnp.float32),
    compiler_params=pltpu.CompilerParams(needs_layout_passes=False),
    scratch_types=[
        pltpu.VMEM((N_NODES,), jnp.float32),
        pltpu.VMEM((N_NODES,), jnp.float32),
        pltpu.VMEM((2, CHUNK), jnp.int32),
        pltpu.VMEM((2, TILE), jnp.int32),
        pltpu.VMEM((CHUNK,), jnp.float32),
        pltpu.VMEM((TILE,), jnp.float32),
        pltpu.SemaphoreType.DMA,
    ],
)
def _sc_edge_gather(s_hbm, edge_hbm, out_hbm,
                    s1_v, s2_v, ei_v, ei2_v, out_v, out2_v, sem):
    wid = lax.axis_index("s") * NC + lax.axis_index("c")
    base = wid * CHUNK
    c1 = pltpu.async_copy(s_hbm.at[0], s1_v, sem)
    c2 = pltpu.async_copy(s_hbm.at[1], s2_v, sem)
    c3 = pltpu.async_copy(edge_hbm.at[:, pl.ds(base, CHUNK)], ei_v, sem)
    c1.wait()
    c2.wait()
    c3.wait()

    @plsc.parallel_loop(0, CHUNK, L, unroll=16)
    def _body(off):
        si = ei_v[0, pl.ds(off, L)]
        di = ei_v[1, pl.ds(off, L)]
        v1 = plsc.load_gather(s1_v, [si])
        v2 = plsc.load_gather(s2_v, [di])
        out_v[pl.ds(off, L)] = v1 + v2

    pltpu.sync_copy(out_v, out_hbm.at[pl.ds(base, CHUNK)])

    @pl.when(wid < N_REM)
    def _rem():
        rbase = REM_BASE + wid * TILE
        pltpu.sync_copy(edge_hbm.at[:, pl.ds(rbase, TILE)], ei2_v)

        @plsc.parallel_loop(0, TILE, L, unroll=8)
        def _body2(off):
            si = ei2_v[0, pl.ds(off, L)]
            di = ei2_v[1, pl.ds(off, L)]
            v1 = plsc.load_gather(s1_v, [si])
            v2 = plsc.load_gather(s2_v, [di])
            out2_v[pl.ds(off, L)] = v1 + v2

        pltpu.sync_copy(out2_v, out_hbm.at[pl.ds(rbase, TILE)])


def kernel(h, edge_index, W, b):
    wmat = W.reshape(2, D_FEAT)
    s = pl.pallas_call(
        _tc_node_scores,
        in_specs=[
            pl.BlockSpec(memory_space=pltpu.SMEM),
            pl.BlockSpec((2, D_FEAT)),
            pl.BlockSpec((N_NODES, D_FEAT)),
        ],
        out_specs=pl.BlockSpec((2, N_NODES)),
        out_shape=jax.ShapeDtypeStruct((2, N_NODES), jnp.float32),
    )(b.reshape(1, 1), wmat, h)
    scores = _sc_edge_gather(s, edge_index)
    return scores.reshape(N_EDGES, 1)


# speedup vs baseline: 62.1121x; 1.2599x over previous
import functools

import jax
import jax.numpy as jnp
from jax import lax
from jax.experimental import pallas as pl
from jax.experimental.pallas import tpu as pltpu
from jax.experimental.pallas import tpu_sc as plsc

N_NODES = 10000
N_EDGES = 320000
D_FEAT = 128
L = 16
NC, NS = 2, 16
NW = NC * NS

TILE = 128
N_TILES = N_EDGES // TILE
T_PER_W = N_TILES // NW
CHUNK = T_PER_W * TILE
REM_BASE = NW * CHUNK
N_REM = N_TILES - NW * T_PER_W


def _tc_node_scores(b_ref, w_ref, h_ref, o_ref):
    rows = lax.broadcasted_iota(jnp.int32, (2, N_NODES), 0)
    bias = jnp.where(rows == 1, b_ref[0, 0], 0.0)
    o_ref[...] = (
        lax.dot_general(
            w_ref[...], h_ref[...],
            dimension_numbers=(((1,), (1,)), ((), ())),
            preferred_element_type=jnp.float32,
        )
        + bias
    )


_sc_mesh = plsc.VectorSubcoreMesh(core_axis_name="c", subcore_axis_name="s")


@functools.partial(
    pl.kernel,
    mesh=_sc_mesh,
    out_type=jax.ShapeDtypeStruct((1, N_EDGES), jnp.float32),
    compiler_params=pltpu.CompilerParams(needs_layout_passes=False),
    scratch_types=[
        pltpu.VMEM((N_NODES,), jnp.float32),
        pltpu.VMEM((N_NODES,), jnp.float32),
        pltpu.VMEM((2, CHUNK), jnp.int32),
        pltpu.VMEM((2, TILE), jnp.int32),
        pltpu.VMEM((CHUNK,), jnp.float32),
        pltpu.VMEM((TILE,), jnp.float32),
        pltpu.SemaphoreType.DMA,
    ],
)
def _sc_edge_gather(s_hbm, edge_hbm, out_hbm,
                    s1_v, s2_v, ei_v, ei2_v, out_v, out2_v, sem):
    wid = lax.axis_index("s") * NC + lax.axis_index("c")
    base = wid * CHUNK
    c1 = pltpu.async_copy(s_hbm.at[0], s1_v, sem)
    c2 = pltpu.async_copy(s_hbm.at[1], s2_v, sem)
    c3 = pltpu.async_copy(edge_hbm.at[:, pl.ds(base, CHUNK)], ei_v, sem)
    c1.wait()
    c2.wait()
    c3.wait()

    @plsc.parallel_loop(0, CHUNK, L, unroll=16)
    def _body(off):
        si = ei_v[0, pl.ds(off, L)]
        di = ei_v[1, pl.ds(off, L)]
        v1 = plsc.load_gather(s1_v, [si])
        v2 = plsc.load_gather(s2_v, [di])
        out_v[pl.ds(off, L)] = v1 + v2

    pltpu.sync_copy(out_v, out_hbm.at[0, pl.ds(base, CHUNK)])

    @pl.when(wid < N_REM)
    def _rem():
        rbase = REM_BASE + wid * TILE
        pltpu.sync_copy(edge_hbm.at[:, pl.ds(rbase, TILE)], ei2_v)

        @plsc.parallel_loop(0, TILE, L, unroll=8)
        def _body2(off):
            si = ei2_v[0, pl.ds(off, L)]
            di = ei2_v[1, pl.ds(off, L)]
            v1 = plsc.load_gather(s1_v, [si])
            v2 = plsc.load_gather(s2_v, [di])
            out2_v[pl.ds(off, L)] = v1 + v2

        pltpu.sync_copy(out2_v, out_hbm.at[0, pl.ds(rbase, TILE)])


def kernel(h, edge_index, W, b):
    wmat = W.reshape(2, D_FEAT)
    s = pl.pallas_call(
        _tc_node_scores,
        in_specs=[
            pl.BlockSpec(memory_space=pltpu.SMEM),
            pl.BlockSpec((2, D_FEAT)),
            pl.BlockSpec((N_NODES, D_FEAT)),
        ],
        out_specs=pl.BlockSpec((2, N_NODES)),
        out_shape=jax.ShapeDtypeStruct((2, N_NODES), jnp.float32),
    )(b.reshape(1, 1), wmat, h)
    scores = _sc_edge_gather(s, edge_index)
    return scores.reshape(N_EDGES, 1)
